# baseline (speedup 1.0000x reference)
import jax
import jax.numpy as jnp
from jax import lax
from jax.experimental import pallas as pl
from jax.experimental.pallas import tpu as pltpu
from jax.experimental.pallas import tpu_sc as plsc

D_MODEL = 1024
MAX_SEQ_LEN = 2048
BATCH = 16
NC = 2
NS = 16
NW = NC * NS
LANES = 16

WCOLS = MAX_SEQ_LEN // NW
GRP = 8
ZROWS = 32


def _sc_body(len_hbm, table_hbm, emb_hbm, pos_hbm,
             len_v, idx_t, idx_z, pos_v, tbl, zbuf, strad, dump,
             rsem, zsem, gsem, wsem, psem):
    wid = lax.axis_index("s") * NC + lax.axis_index("c")
    wcol = wid * WCOLS
    lanes = lax.iota(jnp.int32, LANES)

    for k in range(WCOLS // LANES):
        idx_t[pl.ds(k * LANES, LANES)] = wcol + 1 + k * LANES + lanes
    for k in range(ZROWS // LANES):
        idx_z[pl.ds(k * LANES, LANES)] = jnp.zeros((LANES,), jnp.int32)
    loads = [
        pltpu.async_copy(table_hbm.at[idx_t], tbl, rsem),
        pltpu.async_copy(table_hbm.at[idx_z], zbuf, zsem),
    ]
    pltpu.sync_copy(len_hbm, len_v)

    lens = []
    lenvec = len_v[...]
    for b in range(BATCH):
        len_b = lenvec[b]
        lens.append(len_b)
        len_bv = jnp.full((LANES,), len_b, dtype=jnp.int32)
        for k in range(WCOLS // LANES):
            col = wcol + k * LANES + lanes
            pos_v[b, pl.ds(k * LANES, LANES)] = jnp.where(col < len_bv, col + 1, 0)
        pltpu.async_copy(pos_v.at[b], pos_hbm.at[b, pl.ds(wcol, WCOLS)], psem)

    loads[0].wait()
    loads[1].wait()

    for b in range(BATCH):
        len_b = lens[b]
        full_b = len_b >= wcol + WCOLS
        zero_b = len_b <= wcol

        @pl.when(full_b)
        def _():
            pltpu.async_copy(tbl, emb_hbm.at[b, pl.ds(wcol, WCOLS)], wsem)

        @pl.when(zero_b)
        def _():
            for q in range(WCOLS // ZROWS):
                pltpu.async_copy(
                    zbuf, emb_hbm.at[b, pl.ds(wcol + q * ZROWS, ZROWS)], wsem)

        @pl.when(jnp.logical_not(jnp.logical_or(full_b, zero_b)))
        def _():
            for q in range(WCOLS // GRP):
                lo = wcol + q * GRP
                fullq = len_b >= lo + GRP
                zeroq = len_b <= lo

                @pl.when(fullq)
                def _():
                    pltpu.async_copy(tbl.at[pl.ds(q * GRP, GRP)],
                                     emb_hbm.at[b, pl.ds(lo, GRP)], wsem)

                @pl.when(zeroq)
                def _():
                    pltpu.async_copy(zbuf.at[pl.ds(0, GRP)],
                                     emb_hbm.at[b, pl.ds(lo, GRP)], wsem)

                @pl.when(jnp.logical_not(jnp.logical_or(fullq, zeroq)))
                def _():
                    pltpu.async_copy(
                        table_hbm.at[pos_v.at[b, pl.ds(q * GRP, GRP)]],
                        strad, gsem).wait()
                    pltpu.sync_copy(strad, emb_hbm.at[b, pl.ds(lo, GRP)])
                    pltpu.async_copy(table_hbm.at[pl.ds(0, GRP)], dump, wsem)

    for b in range(BATCH):
        pltpu.make_async_copy(
            table_hbm.at[pl.ds(0, WCOLS)], tbl, wsem).wait()
        pltpu.make_async_copy(
            pos_hbm.at[b, pl.ds(0, WCOLS)], pos_v.at[b], psem).wait()


def kernel(input_len, pe_table):
    mesh = plsc.VectorSubcoreMesh(core_axis_name="c", subcore_axis_name="s")
    f = pl.kernel(
        _sc_body,
        out_type=[
            jax.ShapeDtypeStruct((BATCH, MAX_SEQ_LEN, D_MODEL), jnp.float32),
            jax.ShapeDtypeStruct((BATCH, MAX_SEQ_LEN), jnp.int32),
        ],
        mesh=mesh,
        scratch_types=[
            pltpu.VMEM((LANES,), jnp.int32),
            pltpu.VMEM((WCOLS,), jnp.int32),
            pltpu.VMEM((ZROWS,), jnp.int32),
            pltpu.VMEM((BATCH, WCOLS), jnp.int32),
            pltpu.VMEM((WCOLS, D_MODEL), jnp.float32),
            pltpu.VMEM((ZROWS, D_MODEL), jnp.float32),
            pltpu.VMEM((GRP, D_MODEL), jnp.float32),
            pltpu.VMEM((GRP, D_MODEL), jnp.float32),
            pltpu.SemaphoreType.DMA,
            pltpu.SemaphoreType.DMA,
            pltpu.SemaphoreType.DMA,
            pltpu.SemaphoreType.DMA,
            pltpu.SemaphoreType.DMA,
        ],
    )
    emb, pos = f(input_len, pe_table)
    return emb, pos


# speedup vs baseline: 1.2433x; 1.2433x over previous
import jax
import jax.numpy as jnp
from jax import lax
from jax.experimental import pallas as pl
from jax.experimental.pallas import tpu as pltpu
from jax.experimental.pallas import tpu_sc as plsc

D_MODEL = 1024
MAX_SEQ_LEN = 2048
BATCH = 16
NC = 2
NS = 16
NW = NC * NS
LANES = 16

WCOLS = MAX_SEQ_LEN // NW
GRP = 8
ZROWS = 16


def _sc_body(len_hbm, table_hbm, emb_hbm, pos_hbm,
             len_v, idx_t, idx_z, pos_v, tbl, zbuf, strad, dump,
             rsem, zsem, gsem, wsem, psem):
    wid = lax.axis_index("s") * NC + lax.axis_index("c")
    wcol = wid * WCOLS
    lanes = lax.iota(jnp.int32, LANES)

    for k in range(WCOLS // LANES):
        idx_t[pl.ds(k * LANES, LANES)] = wcol + 1 + k * LANES + lanes
    for k in range(ZROWS // LANES):
        idx_z[pl.ds(k * LANES, LANES)] = jnp.zeros((LANES,), jnp.int32)
    loads = [
        pltpu.async_copy(table_hbm.at[idx_t], tbl, rsem),
        pltpu.async_copy(table_hbm.at[idx_z], zbuf, zsem),
    ]
    pltpu.sync_copy(len_hbm, len_v)

    lens = []
    lenvec = len_v[...]
    for b in range(BATCH):
        len_b = lenvec[b]
        lens.append(len_b)
        len_bv = jnp.full((LANES,), len_b, dtype=jnp.int32)
        for k in range(WCOLS // LANES):
            col = wcol + k * LANES + lanes
            pos_v[b, pl.ds(k * LANES, LANES)] = jnp.where(col < len_bv, col + 1, 0)
        pltpu.async_copy(pos_v.at[b], pos_hbm.at[b, pl.ds(wcol, WCOLS)], psem)

    loads[0].wait()
    loads[1].wait()

    for b in range(BATCH):
        len_b = lens[b]
        full_b = len_b >= wcol + WCOLS
        zero_b = len_b <= wcol

        @pl.when(full_b)
        def _():
            pltpu.async_copy(tbl, emb_hbm.at[b, pl.ds(wcol, WCOLS)], wsem)

        @pl.when(zero_b)
        def _():
            for q in range(WCOLS // ZROWS):
                pltpu.async_copy(
                    zbuf, emb_hbm.at[b, pl.ds(wcol + q * ZROWS, ZROWS)], wsem)

        @pl.when(jnp.logical_not(jnp.logical_or(full_b, zero_b)))
        def _():
            for q in range(WCOLS // GRP):
                lo = wcol + q * GRP
                fullq = len_b >= lo + GRP
                zeroq = len_b <= lo

                @pl.when(fullq)
                def _():
                    pltpu.async_copy(tbl.at[pl.ds(q * GRP, GRP)],
                                     emb_hbm.at[b, pl.ds(lo, GRP)], wsem)

                @pl.when(zeroq)
                def _():
                    pltpu.async_copy(zbuf.at[pl.ds(0, GRP)],
                                     emb_hbm.at[b, pl.ds(lo, GRP)], wsem)

                @pl.when(jnp.logical_not(jnp.logical_or(fullq, zeroq)))
                def _():
                    pltpu.async_copy(
                        table_hbm.at[pos_v.at[b, pl.ds(q * GRP, GRP)]],
                        strad, gsem).wait()
                    pltpu.sync_copy(strad, emb_hbm.at[b, pl.ds(lo, GRP)])
                    pltpu.async_copy(table_hbm.at[pl.ds(0, GRP)], dump, wsem)

    for b in range(BATCH):
        pltpu.make_async_copy(
            table_hbm.at[pl.ds(0, WCOLS)], tbl, wsem).wait()
        pltpu.make_async_copy(
            pos_hbm.at[b, pl.ds(0, WCOLS)], pos_v.at[b], psem).wait()


def kernel(input_len, pe_table):
    mesh = plsc.VectorSubcoreMesh(core_axis_name="c", subcore_axis_name="s")
    f = pl.kernel(
        _sc_body,
        out_type=[
            jax.ShapeDtypeStruct((BATCH, MAX_SEQ_LEN, D_MODEL), jnp.float32),
            jax.ShapeDtypeStruct((BATCH, MAX_SEQ_LEN), jnp.int32),
        ],
        mesh=mesh,
        scratch_types=[
            pltpu.VMEM((LANES,), jnp.int32),
            pltpu.VMEM((WCOLS,), jnp.int32),
            pltpu.VMEM((ZROWS,), jnp.int32),
            pltpu.VMEM((BATCH, WCOLS), jnp.int32),
            pltpu.VMEM((WCOLS, D_MODEL), jnp.float32),
            pltpu.VMEM((ZROWS, D_MODEL), jnp.float32),
            pltpu.VMEM((GRP, D_MODEL), jnp.float32),
            pltpu.VMEM((GRP, D_MODEL), jnp.float32),
            pltpu.SemaphoreType.DMA,
            pltpu.SemaphoreType.DMA,
            pltpu.SemaphoreType.DMA,
            pltpu.SemaphoreType.DMA,
            pltpu.SemaphoreType.DMA,
        ],
    )
    emb, pos = f(input_len, pe_table)
    return emb, pos


# speedup vs baseline: 1.2530x; 1.0078x over previous
import jax
import jax.numpy as jnp
from jax import lax
from jax.experimental import pallas as pl
from jax.experimental.pallas import tpu as pltpu
from jax.experimental.pallas import tpu_sc as plsc

D_MODEL = 1024
MAX_SEQ_LEN = 2048
BATCH = 16
NC = 2
NS = 16
NW = NC * NS
LANES = 16

WCOLS = MAX_SEQ_LEN // NW
GRP = 8
ZROWS = 16


def _sc_body(len_hbm, table_hbm, emb_hbm, pos_hbm,
             len_v, idx_t, idx_z, pos_v, tbl, zbuf, strad, dump,
             rsem, zsem, gsem, wsem, psem):
    wid = lax.axis_index("s") * NC + lax.axis_index("c")
    wcol = wid * WCOLS
    lanes = lax.iota(jnp.int32, LANES)

    for k in range(WCOLS // LANES):
        idx_t[pl.ds(k * LANES, LANES)] = wcol + 1 + k * LANES + lanes
    for k in range(ZROWS // LANES):
        idx_z[pl.ds(k * LANES, LANES)] = jnp.zeros((LANES,), jnp.int32)
    loads = [
        pltpu.async_copy(table_hbm.at[idx_t], tbl, rsem),
        pltpu.async_copy(table_hbm.at[idx_z], zbuf, zsem),
    ]
    pltpu.sync_copy(len_hbm, len_v)

    lens = []
    lenvec = len_v[...]
    for b in range(BATCH):
        len_b = lenvec[b]
        lens.append(len_b)
        len_bv = jnp.full((LANES,), len_b, dtype=jnp.int32)
        for k in range(WCOLS // LANES):
            col = wcol + k * LANES + lanes
            pos_v[b, pl.ds(k * LANES, LANES)] = jnp.where(col < len_bv, col + 1, 0)
        pltpu.async_copy(pos_v.at[b], pos_hbm.at[b, pl.ds(wcol, WCOLS)], psem)

    loads[0].wait()
    loads[1].wait()

    for b in range(BATCH):
        len_b = lens[b]
        full_b = len_b >= wcol + WCOLS
        zero_b = len_b <= wcol

        @pl.when(full_b)
        def _():
            for q in range(WCOLS // ZROWS):
                pltpu.async_copy(
                    tbl.at[pl.ds(q * ZROWS, ZROWS)],
                    emb_hbm.at[b, pl.ds(wcol + q * ZROWS, ZROWS)], wsem)

        @pl.when(zero_b)
        def _():
            for q in range(WCOLS // ZROWS):
                pltpu.async_copy(
                    zbuf, emb_hbm.at[b, pl.ds(wcol + q * ZROWS, ZROWS)], wsem)

        @pl.when(jnp.logical_not(jnp.logical_or(full_b, zero_b)))
        def _():
            for q in range(WCOLS // GRP):
                lo = wcol + q * GRP
                fullq = len_b >= lo + GRP
                zeroq = len_b <= lo

                @pl.when(fullq)
                def _():
                    pltpu.async_copy(tbl.at[pl.ds(q * GRP, GRP)],
                                     emb_hbm.at[b, pl.ds(lo, GRP)], wsem)

                @pl.when(zeroq)
                def _():
                    pltpu.async_copy(zbuf.at[pl.ds(0, GRP)],
                                     emb_hbm.at[b, pl.ds(lo, GRP)], wsem)

                @pl.when(jnp.logical_not(jnp.logical_or(fullq, zeroq)))
                def _():
                    pltpu.async_copy(
                        table_hbm.at[pos_v.at[b, pl.ds(q * GRP, GRP)]],
                        strad, gsem).wait()
                    pltpu.sync_copy(strad, emb_hbm.at[b, pl.ds(lo, GRP)])
                    pltpu.async_copy(table_hbm.at[pl.ds(0, GRP)], dump, wsem)

    for b in range(BATCH):
        pltpu.make_async_copy(
            table_hbm.at[pl.ds(0, WCOLS)], tbl, wsem).wait()
        pltpu.make_async_copy(
            pos_hbm.at[b, pl.ds(0, WCOLS)], pos_v.at[b], psem).wait()


def kernel(input_len, pe_table):
    mesh = plsc.VectorSubcoreMesh(core_axis_name="c", subcore_axis_name="s")
    f = pl.kernel(
        _sc_body,
        out_type=[
            jax.ShapeDtypeStruct((BATCH, MAX_SEQ_LEN, D_MODEL), jnp.float32),
            jax.ShapeDtypeStruct((BATCH, MAX_SEQ_LEN), jnp.int32),
        ],
        mesh=mesh,
        scratch_types=[
            pltpu.VMEM((LANES,), jnp.int32),
            pltpu.VMEM((WCOLS,), jnp.int32),
            pltpu.VMEM((ZROWS,), jnp.int32),
            pltpu.VMEM((BATCH, WCOLS), jnp.int32),
            pltpu.VMEM((WCOLS, D_MODEL), jnp.float32),
            pltpu.VMEM((ZROWS, D_MODEL), jnp.float32),
            pltpu.VMEM((GRP, D_MODEL), jnp.float32),
            pltpu.VMEM((GRP, D_MODEL), jnp.float32),
            pltpu.SemaphoreType.DMA,
            pltpu.SemaphoreType.DMA,
            pltpu.SemaphoreType.DMA,
            pltpu.SemaphoreType.DMA,
            pltpu.SemaphoreType.DMA,
        ],
    )
    emb, pos = f(input_len, pe_table)
    return emb, pos


# speedup vs baseline: 1.2714x; 1.0147x over previous
import jax
import jax.numpy as jnp
from jax import lax
from jax.experimental import pallas as pl
from jax.experimental.pallas import tpu as pltpu
from jax.experimental.pallas import tpu_sc as plsc

D_MODEL = 1024
MAX_SEQ_LEN = 2048
BATCH = 16
NC = 2
NS = 16
NW = NC * NS
LANES = 16

WCOLS = MAX_SEQ_LEN // NW
GRP = 8
ZROWS = 16


def _sc_body(len_hbm, table_hbm, emb_hbm, pos_hbm,
             len_v, idx_t, idx_z, pos_v, tbl, zbuf, strad, dump,
             rsem, zsem, gsem, wsem, psem):
    wid = lax.axis_index("s") * NC + lax.axis_index("c")
    wcol = wid * WCOLS
    lanes = lax.iota(jnp.int32, LANES)

    for k in range(WCOLS // LANES):
        idx_t[pl.ds(k * LANES, LANES)] = wcol + 1 + k * LANES + lanes
    for k in range(ZROWS // LANES):
        idx_z[pl.ds(k * LANES, LANES)] = jnp.zeros((LANES,), jnp.int32)
    loads = [
        pltpu.async_copy(table_hbm.at[idx_t], tbl, rsem),
        pltpu.async_copy(table_hbm.at[idx_z], zbuf, zsem),
    ]
    pltpu.sync_copy(len_hbm, len_v)

    lens = []
    lenvec = len_v[...]
    for b in range(BATCH):
        len_b = lenvec[b]
        lens.append(len_b)
        len_bv = jnp.full((LANES,), len_b, dtype=jnp.int32)
        for k in range(WCOLS // LANES):
            col = wcol + k * LANES + lanes
            pos_v[b, pl.ds(k * LANES, LANES)] = jnp.where(col < len_bv, col + 1, 0)
        pltpu.async_copy(pos_v.at[b], pos_hbm.at[b, pl.ds(wcol, WCOLS)], psem)

    loads[0].wait()
    loads[1].wait()

    for b in range(BATCH):
        len_b = lens[b]
        full_b = len_b >= wcol + WCOLS
        zero_b = len_b <= wcol

        @pl.when(full_b)
        def _():
            for q in range(WCOLS // GRP):
                pltpu.async_copy(
                    tbl.at[pl.ds(q * GRP, GRP)],
                    emb_hbm.at[b, pl.ds(wcol + q * GRP, GRP)], wsem)

        @pl.when(zero_b)
        def _():
            for q in range(WCOLS // GRP):
                pltpu.async_copy(
                    zbuf.at[pl.ds(0, GRP)],
                    emb_hbm.at[b, pl.ds(wcol + q * GRP, GRP)], wsem)

        @pl.when(jnp.logical_not(jnp.logical_or(full_b, zero_b)))
        def _():
            for q in range(WCOLS // GRP):
                lo = wcol + q * GRP
                fullq = len_b >= lo + GRP
                zeroq = len_b <= lo

                @pl.when(fullq)
                def _():
                    pltpu.async_copy(tbl.at[pl.ds(q * GRP, GRP)],
                                     emb_hbm.at[b, pl.ds(lo, GRP)], wsem)

                @pl.when(zeroq)
                def _():
                    pltpu.async_copy(zbuf.at[pl.ds(0, GRP)],
                                     emb_hbm.at[b, pl.ds(lo, GRP)], wsem)

                @pl.when(jnp.logical_not(jnp.logical_or(fullq, zeroq)))
                def _():
                    pltpu.async_copy(
                        table_hbm.at[pos_v.at[b, pl.ds(q * GRP, GRP)]],
                        strad, gsem).wait()
                    pltpu.sync_copy(strad, emb_hbm.at[b, pl.ds(lo, GRP)])
                    pltpu.async_copy(table_hbm.at[pl.ds(0, GRP)], dump, wsem)

    for b in range(BATCH):
        pltpu.make_async_copy(
            table_hbm.at[pl.ds(0, WCOLS)], tbl, wsem).wait()
        pltpu.make_async_copy(
            pos_hbm.at[b, pl.ds(0, WCOLS)], pos_v.at[b], psem).wait()


def kernel(input_len, pe_table):
    mesh = plsc.VectorSubcoreMesh(core_axis_name="c", subcore_axis_name="s")
    f = pl.kernel(
        _sc_body,
        out_type=[
            jax.ShapeDtypeStruct((BATCH, MAX_SEQ_LEN, D_MODEL), jnp.float32),
            jax.ShapeDtypeStruct((BATCH, MAX_SEQ_LEN), jnp.int32),
        ],
        mesh=mesh,
        scratch_types=[
            pltpu.VMEM((LANES,), jnp.int32),
            pltpu.VMEM((WCOLS,), jnp.int32),
            pltpu.VMEM((ZROWS,), jnp.int32),
            pltpu.VMEM((BATCH, WCOLS), jnp.int32),
            pltpu.VMEM((WCOLS, D_MODEL), jnp.float32),
            pltpu.VMEM((ZROWS, D_MODEL), jnp.float32),
            pltpu.VMEM((GRP, D_MODEL), jnp.float32),
            pltpu.VMEM((GRP, D_MODEL), jnp.float32),
            pltpu.SemaphoreType.DMA,
            pltpu.SemaphoreType.DMA,
            pltpu.SemaphoreType.DMA,
            pltpu.SemaphoreType.DMA,
            pltpu.SemaphoreType.DMA,
        ],
    )
    emb, pos = f(input_len, pe_table)
    return emb, pos
